# baseline (speedup 1.0000x reference)
import functools

import jax
import jax.numpy as jnp
from jax import lax
from jax.experimental import pallas as pl
from jax.experimental.pallas import tpu as pltpu

_N = 5000
_ROWS = 40
_COLS = 128
_PAD = _ROWS * _COLS
_NB = _PAD // _COLS
_THR = 0.5


def _f32(x):
    return x.astype(jnp.float32)


def _recombine(nibs):
    acc = nibs[0].astype(jnp.int32)
    for nb in nibs[1:]:
        acc = (acc << 4) | nb.astype(jnp.int32)
    return lax.bitcast_convert_type(acc, jnp.float32)


def _nms_v5(C_ref, s_ref, keep_ref, sfr_ref, sfrT_ref, rank_ref, sC_ref,
            sCC_ref, rows_ref, supp_ref, kept_ref, keptT_ref):
    f32 = jnp.float32
    shape = (_ROWS, _COLS)
    lin = (
        lax.broadcasted_iota(jnp.int32, shape, 0) * _COLS
        + lax.broadcasted_iota(jnp.int32, shape, 1)
    )
    valid = lin < _N

    s = s_ref[...]
    neg_inf = f32(-jnp.inf)
    pos_inf = f32(jnp.inf)
    s_for_max = jnp.where(valid, s, neg_inf)
    first_max = jnp.max(s_for_max)
    first_min = jnp.min(jnp.where(valid, s, pos_inf))
    init_count = jnp.sum((valid & (s < first_max)).astype(jnp.int32))

    col_iota_i = lax.broadcasted_iota(jnp.int32, (_COLS, 1), 0)
    row_iota_i = lax.broadcasted_iota(jnp.int32, (1, _COLS), 1)
    col_iota_f = _f32(col_iota_i)
    row_iota_f = _f32(row_iota_i)

    sfr_ref[...] = s_for_max

    def transp_j(jr, _):
        row = sfr_ref[pl.ds(jr, 1), :]
        sfrT_ref[pl.ds(jr * _COLS, _COLS), :] = jnp.swapaxes(row, 0, 1)
        return 0

    lax.fori_loop(0, _NB, transp_j, 0)

    def rank_i_one(ir):
        irow = sfr_ref[pl.ds(ir, 1), :]
        iix = row_iota_f + _f32(ir) * 128.0

        def rank_j(ju, acc):
            a0, a1 = acc
            for u in range(8):
                jr = ju * 8 + u
                jcol = sfrT_ref[pl.ds(jr * _COLS, _COLS), :]
                jix = col_iota_f + _f32(jr) * 128.0
                cmpv = (jcol > irow) | ((jcol == irow) & (jix < iix))
                c = jnp.sum(_f32(cmpv), axis=0, keepdims=True)
                if u % 2 == 0:
                    a0 = a0 + c
                else:
                    a1 = a1 + c
            return a0, a1

        z2 = jnp.zeros((1, _COLS), f32)
        c0, c1 = lax.fori_loop(0, _NB // 8, rank_j, (z2, z2))
        cnt = c0 + c1
        rank_ref[pl.ds(ir, 1), :] = cnt

    def rank_i(iu, _):
        rank_i_one(iu * 2)
        rank_i_one(iu * 2 + 1)
        return 0

    lax.fori_loop(0, _NB // 2, rank_i, 0)

    def perm_k_one(kb):
        kcol = col_iota_f + _f32(kb) * 128.0

        def perm_acc(iu, acc):
            for u in range(8):
                ir = iu * 8 + u
                rrow = rank_ref[pl.ds(ir, 1), :]
                pt = _f32(rrow == kcol)
                cb = C_ref[pl.ds(ir * _COLS, _COLS), :]
                acc = acc + jnp.dot(pt, cb, preferred_element_type=f32)
            return acc

        blk = lax.fori_loop(0, _NB // 8, perm_acc,
                            jnp.zeros((_COLS, 40), f32))
        sC_ref[pl.ds(kb * _COLS, _COLS), :] = blk

    def perm_k(ku, _):
        perm_k_one(ku * 2)
        perm_k_one(ku * 2 + 1)
        return 0

    lax.fori_loop(0, _NB // 2, perm_k, 0)

    def prep_b(bb, _):
        blk = sC_ref[pl.ds(bb * _COLS, _COLS), :]

        def rc(a):
            return _recombine([blk[:, a * 8 + t:a * 8 + t + 1]
                               for t in range(8)])

        x1 = rc(0)
        y1 = rc(1)
        x2 = rc(2)
        y2 = rc(3)
        ss = rc(4)
        ar = (x2 - x1) * (y2 - y1)
        z = jnp.zeros((_COLS, 1), f32)
        sCC_ref[pl.ds(bb * _COLS, _COLS), :] = jnp.concatenate(
            [x1, y1, x2, y2, ar, ss, z, z], axis=1)
        for a, v in enumerate((x1, y1, x2, y2, ar, ss)):
            rows_ref[pl.ds(a * _NB + bb, 1), :] = jnp.swapaxes(v, 0, 1)
        return 0

    lax.fori_loop(0, _NB, prep_b, 0)

    def get_cols(bb):
        blk = sCC_ref[pl.ds(bb * _COLS, _COLS), :]
        return (blk[:, 0:1], blk[:, 1:2], blk[:, 2:3], blk[:, 3:4],
                blk[:, 4:5])

    def get_rows(bb):
        return tuple(rows_ref[pl.ds(a * _NB + bb, 1), :] for a in range(5))

    def iou_gt(cols, rows):
        x1c, y1c, x2c, y2c, ac = cols
        x1r, y1r, x2r, y2r, ar = rows
        xx = jnp.minimum(x2c, x2r) - jnp.maximum(x1c, x1r)
        yy = jnp.minimum(y2c, y2r) - jnp.maximum(y1c, y1r)
        inter = jnp.maximum(xx, 0.0) * jnp.maximum(yy, 0.0)
        iou = inter / ((ac + ar) - inter)
        return iou > _THR

    kept_ref[...] = jnp.zeros(shape, f32)

    def init_supp(bb, _):
        ss = rows_ref[pl.ds(5 * _NB + bb, 1), :]
        klin = row_iota_i + bb * _COLS
        supp = (klin >= _N) | ((ss == first_max) & (klin > 0))
        supp_ref[pl.ds(bb, 1), :] = _f32(supp)
        return 0

    lax.fori_loop(0, _NB, init_supp, 0)

    def scan_b(bb, _):
        cols = get_cols(bb)
        rows = get_rows(bb)
        m = _f32(iou_gt(cols, rows) & (col_iota_f < row_iota_f))

        dead0 = supp_ref[pl.ds(bb, 1), :]
        kept0 = jnp.zeros((1, _COLS), f32)
        rem0 = jnp.sum((1.0 - dead0))

        def round_cond(st):
            return st[2] > 0.5

        def round_body(st):
            dead, kept, _ = st
            deadc = jnp.swapaxes(dead, 0, 1)
            keptc = jnp.swapaxes(kept, 0, 1)
            threat = jnp.max(m * (1.0 - deadc), axis=0, keepdims=True)
            kill = jnp.max(m * keptc, axis=0, keepdims=True)
            unknown = (1.0 - dead) * (1.0 - kept)
            newkept = unknown * (1.0 - threat)
            newdead = unknown * kill
            kept = kept + newkept
            dead = dead + newdead
            rem = jnp.sum((1.0 - dead) * (1.0 - kept))
            return dead, kept, rem

        _, kept, _ = lax.while_loop(round_cond, round_body,
                                    (dead0, kept0, rem0))
        kept_ref[pl.ds(bb, 1), :] = kept
        keptc = jnp.swapaxes(kept, 0, 1)

        def sweep_tile(cc):
            kill2 = jnp.max(_f32(iou_gt(cols, get_rows(cc))) * keptc,
                            axis=0, keepdims=True)
            supp_ref[pl.ds(cc, 1), :] = jnp.maximum(
                supp_ref[pl.ds(cc, 1), :], kill2)

        trip = _NB - 1 - bb

        def sweep_c(p, _):
            cc = bb + 1 + p * 8
            for u in range(8):
                sweep_tile(cc + u)
            return 0

        lax.fori_loop(0, trip // 8, sweep_c, 0)
        rem = trip % 8
        for t in range(1, 8):

            @pl.when(rem >= t)
            def _(t=t):
                sweep_tile(_NB - t)

        return 0

    lax.fori_loop(0, _NB, scan_b, 0)

    def transp_k(kb, _):
        row = kept_ref[pl.ds(kb, 1), :]
        keptT_ref[pl.ds(kb * _COLS, _COLS), :] = jnp.swapaxes(row, 0, 1)
        return 0

    lax.fori_loop(0, _NB, transp_k, 0)

    gate = _f32(init_count > 0)

    def unperm_i_one(ir):
        rrow = rank_ref[pl.ds(ir, 1), :]

        def unperm_acc(ku, acc):
            a0, a1 = acc
            for u in range(8):
                kb = ku * 8 + u
                kcol = col_iota_f + _f32(kb) * 128.0
                ksc = keptT_ref[pl.ds(kb * _COLS, _COLS), :]
                hit = jnp.where(rrow == kcol, ksc, 0.0)
                c = jnp.sum(hit, axis=0, keepdims=True)
                if u % 2 == 0:
                    a0 = a0 + c
                else:
                    a1 = a1 + c
            return a0, a1

        z2 = jnp.zeros((1, _COLS), f32)
        r0, r1 = lax.fori_loop(0, _NB // 8, unperm_acc, (z2, z2))
        row = r0 + r1
        keep_ref[pl.ds(ir, 1), :] = row * gate

    def unperm_i(iu, _):
        unperm_i_one(iu * 2)
        unperm_i_one(iu * 2 + 1)
        return 0

    lax.fori_loop(0, _NB // 2, unperm_i, 0)


def _split4(x):
    bits = lax.bitcast_convert_type(x, jnp.int32)
    return [((bits >> sh) & 0xF).astype(jnp.float32)
            for sh in range(28, -1, -4)]


@functools.partial(jax.jit)
def kernel(boxes, scores):
    pad = _PAD - _N

    s = jnp.pad(scores, (0, pad)).reshape(_ROWS, _COLS)

    cols = []
    for arr in (boxes[:, 0], boxes[:, 1], boxes[:, 2], boxes[:, 3], scores):
        cols.extend(_split4(jnp.pad(arr, (0, pad))))
    C = jnp.stack(cols, axis=1)

    f32 = jnp.float32
    keep = pl.pallas_call(
        _nms_v5,
        out_shape=jax.ShapeDtypeStruct((_ROWS, _COLS), f32),
        scratch_shapes=[
            pltpu.VMEM((_ROWS, _COLS), f32),
            pltpu.VMEM((_PAD, 1), f32),
            pltpu.VMEM((_ROWS, _COLS), f32),
            pltpu.VMEM((_PAD, 40), f32),
            pltpu.VMEM((_PAD, 8), f32),
            pltpu.VMEM((6 * _NB, _COLS), f32),
            pltpu.VMEM((_ROWS, _COLS), f32),
            pltpu.VMEM((_ROWS, _COLS), f32),
            pltpu.VMEM((_PAD, 1), f32),
        ],
    )(C, s)

    m = keep.reshape(_PAD)[:_N]
    return jnp.concatenate([boxes, scores[:, None]], axis=1) * m[:, None]


# speedup vs baseline: 1.0232x; 1.0232x over previous
import functools

import jax
import jax.numpy as jnp
from jax import lax
from jax.experimental import pallas as pl
from jax.experimental.pallas import tpu as pltpu

_N = 5000
_ROWS = 40
_COLS = 128
_PAD = _ROWS * _COLS
_NB = _PAD // _COLS
_THR = 0.5


def _f32(x):
    return x.astype(jnp.float32)


def _recombine(nibs):
    acc = nibs[0].astype(jnp.int32)
    for nb in nibs[1:]:
        acc = (acc << 4) | nb.astype(jnp.int32)
    return lax.bitcast_convert_type(acc, jnp.float32)


def _nms_v5(C_ref, s_ref, keep_ref, sfr_ref, sfrT_ref, rank_ref, sC_ref,
            sCC_ref, rows_ref, supp_ref, kept_ref, keptT_ref):
    f32 = jnp.float32
    shape = (_ROWS, _COLS)
    lin = (
        lax.broadcasted_iota(jnp.int32, shape, 0) * _COLS
        + lax.broadcasted_iota(jnp.int32, shape, 1)
    )
    valid = lin < _N

    s = s_ref[...]
    neg_inf = f32(-jnp.inf)
    pos_inf = f32(jnp.inf)
    s_for_max = jnp.where(valid, s, neg_inf)
    first_max = jnp.max(s_for_max)
    first_min = jnp.min(jnp.where(valid, s, pos_inf))
    init_count = jnp.sum((valid & (s < first_max)).astype(jnp.int32))

    col_iota_i = lax.broadcasted_iota(jnp.int32, (_COLS, 1), 0)
    row_iota_i = lax.broadcasted_iota(jnp.int32, (1, _COLS), 1)
    col_iota_f = _f32(col_iota_i)
    row_iota_f = _f32(row_iota_i)

    sfr_ref[...] = s_for_max

    def transp_j(jr, _):
        row = sfr_ref[pl.ds(jr, 1), :]
        sfrT_ref[pl.ds(jr * _COLS, _COLS), :] = jnp.swapaxes(row, 0, 1)
        return 0

    lax.fori_loop(0, _NB, transp_j, 0)

    def rank_i_one(ir):
        irow = sfr_ref[pl.ds(ir, 1), :]
        iix = row_iota_f + _f32(ir) * 128.0

        def rank_j(ju, acc):
            a0, a1 = acc
            for u in range(8):
                jr = ju * 8 + u
                jcol = sfrT_ref[pl.ds(jr * _COLS, _COLS), :]
                jix = col_iota_f + _f32(jr) * 128.0
                cmpv = (jcol > irow) | ((jcol == irow) & (jix < iix))
                c = jnp.sum(_f32(cmpv), axis=0, keepdims=True)
                if u % 2 == 0:
                    a0 = a0 + c
                else:
                    a1 = a1 + c
            return a0, a1

        z2 = jnp.zeros((1, _COLS), f32)
        c0, c1 = lax.fori_loop(0, _NB // 8, rank_j, (z2, z2))
        cnt = c0 + c1
        rank_ref[pl.ds(ir, 1), :] = cnt

    def rank_i(iu, _):
        rank_i_one(iu * 2)
        rank_i_one(iu * 2 + 1)
        return 0

    lax.fori_loop(0, _NB // 2, rank_i, 0)

    def perm_k_one(kb):
        kcol = col_iota_f + _f32(kb) * 128.0

        def perm_acc(iu, acc):
            for u in range(8):
                ir = iu * 8 + u
                rrow = rank_ref[pl.ds(ir, 1), :]
                pt = _f32(rrow == kcol)
                cb = C_ref[pl.ds(ir * _COLS, _COLS), :]
                acc = acc + jnp.dot(pt, cb, preferred_element_type=f32)
            return acc

        blk = lax.fori_loop(0, _NB // 8, perm_acc,
                            jnp.zeros((_COLS, 40), f32))
        sC_ref[pl.ds(kb * _COLS, _COLS), :] = blk

    def perm_k(ku, _):
        perm_k_one(ku * 2)
        perm_k_one(ku * 2 + 1)
        return 0

    lax.fori_loop(0, _NB // 2, perm_k, 0)

    def prep_b(bb, _):
        blk = sC_ref[pl.ds(bb * _COLS, _COLS), :]

        def rc(a):
            return _recombine([blk[:, a * 8 + t:a * 8 + t + 1]
                               for t in range(8)])

        x1 = rc(0)
        y1 = rc(1)
        x2 = rc(2)
        y2 = rc(3)
        ss = rc(4)
        ar = (x2 - x1) * (y2 - y1)
        z = jnp.zeros((_COLS, 1), f32)
        sCC_ref[pl.ds(bb * _COLS, _COLS), :] = jnp.concatenate(
            [x1, y1, x2, y2, ar, ss, z, z], axis=1)
        for a, v in enumerate((x1, y1, x2, y2, ar, ss)):
            rows_ref[pl.ds(a * _NB + bb, 1), :] = jnp.swapaxes(v, 0, 1)
        return 0

    lax.fori_loop(0, _NB, prep_b, 0)

    def get_cols(bb):
        blk = sCC_ref[pl.ds(bb * _COLS, _COLS), :]
        return (blk[:, 0:1], blk[:, 1:2], blk[:, 2:3], blk[:, 3:4],
                blk[:, 4:5])

    def get_rows(bb):
        return tuple(rows_ref[pl.ds(a * _NB + bb, 1), :] for a in range(5))

    def iou_gt(cols, rows):
        x1c, y1c, x2c, y2c, ac = cols
        x1r, y1r, x2r, y2r, ar = rows
        xx = jnp.minimum(x2c, x2r) - jnp.maximum(x1c, x1r)
        yy = jnp.minimum(y2c, y2r) - jnp.maximum(y1c, y1r)
        inter = jnp.maximum(xx, 0.0) * jnp.maximum(yy, 0.0)
        iou = inter / ((ac + ar) - inter)
        return iou > _THR

    kept_ref[...] = jnp.zeros(shape, f32)

    def init_supp(bb, _):
        ss = rows_ref[pl.ds(5 * _NB + bb, 1), :]
        klin = row_iota_i + bb * _COLS
        supp = (klin >= _N) | ((ss == first_max) & (klin > 0))
        supp_ref[pl.ds(bb, 1), :] = _f32(supp)
        return 0

    lax.fori_loop(0, _NB, init_supp, 0)

    def scan_b(bb, _):
        cols = get_cols(bb)
        rows = get_rows(bb)
        m = _f32(iou_gt(cols, rows) & (col_iota_f < row_iota_f))

        dead0 = supp_ref[pl.ds(bb, 1), :]
        kept0 = jnp.zeros((1, _COLS), f32)
        rem0 = jnp.sum((1.0 - dead0))

        def round_cond(st):
            return st[2] > 0.5

        def round_body(st):
            dead, kept, _ = st
            deadc = jnp.swapaxes(dead, 0, 1)
            keptc = jnp.swapaxes(kept, 0, 1)
            threat = jnp.max(m * (1.0 - deadc), axis=0, keepdims=True)
            kill = jnp.max(m * keptc, axis=0, keepdims=True)
            unknown = (1.0 - dead) * (1.0 - kept)
            newkept = unknown * (1.0 - threat)
            newdead = unknown * kill
            kept = kept + newkept
            dead = dead + newdead
            rem = jnp.sum((1.0 - dead) * (1.0 - kept))
            return dead, kept, rem

        _, kept, _ = lax.while_loop(round_cond, round_body,
                                    (dead0, kept0, rem0))
        kept_ref[pl.ds(bb, 1), :] = kept
        keptc = jnp.swapaxes(kept, 0, 1)

        def sweep_tile(cc):
            kill2 = jnp.max(_f32(iou_gt(cols, get_rows(cc))) * keptc,
                            axis=0, keepdims=True)
            supp_ref[pl.ds(cc, 1), :] = jnp.maximum(
                supp_ref[pl.ds(cc, 1), :], kill2)

        trip = _NB - 1 - bb

        def sweep_c(p, _):
            cc = bb + 1 + p * 4
            for u in range(4):
                sweep_tile(cc + u)
            return 0

        lax.fori_loop(0, trip // 4, sweep_c, 0)
        rem = trip % 4
        for t in range(1, 4):

            @pl.when(rem >= t)
            def _(t=t):
                sweep_tile(_NB - t)

        return 0

    lax.fori_loop(0, _NB, scan_b, 0)

    def transp_k(kb, _):
        row = kept_ref[pl.ds(kb, 1), :]
        keptT_ref[pl.ds(kb * _COLS, _COLS), :] = jnp.swapaxes(row, 0, 1)
        return 0

    lax.fori_loop(0, _NB, transp_k, 0)

    gate = _f32(init_count > 0)

    def unperm_i_one(ir):
        rrow = rank_ref[pl.ds(ir, 1), :]

        def unperm_acc(ku, acc):
            a0, a1 = acc
            for u in range(8):
                kb = ku * 8 + u
                kcol = col_iota_f + _f32(kb) * 128.0
                ksc = keptT_ref[pl.ds(kb * _COLS, _COLS), :]
                hit = jnp.where(rrow == kcol, ksc, 0.0)
                c = jnp.sum(hit, axis=0, keepdims=True)
                if u % 2 == 0:
                    a0 = a0 + c
                else:
                    a1 = a1 + c
            return a0, a1

        z2 = jnp.zeros((1, _COLS), f32)
        r0, r1 = lax.fori_loop(0, _NB // 8, unperm_acc, (z2, z2))
        row = r0 + r1
        keep_ref[pl.ds(ir, 1), :] = row * gate

    def unperm_i(iu, _):
        unperm_i_one(iu * 2)
        unperm_i_one(iu * 2 + 1)
        return 0

    lax.fori_loop(0, _NB // 2, unperm_i, 0)


def _split4(x):
    bits = lax.bitcast_convert_type(x, jnp.int32)
    return [((bits >> sh) & 0xF).astype(jnp.float32)
            for sh in range(28, -1, -4)]


@functools.partial(jax.jit)
def kernel(boxes, scores):
    pad = _PAD - _N

    s = jnp.pad(scores, (0, pad)).reshape(_ROWS, _COLS)

    cols = []
    for arr in (boxes[:, 0], boxes[:, 1], boxes[:, 2], boxes[:, 3], scores):
        cols.extend(_split4(jnp.pad(arr, (0, pad))))
    C = jnp.stack(cols, axis=1)

    f32 = jnp.float32
    keep = pl.pallas_call(
        _nms_v5,
        out_shape=jax.ShapeDtypeStruct((_ROWS, _COLS), f32),
        scratch_shapes=[
            pltpu.VMEM((_ROWS, _COLS), f32),
            pltpu.VMEM((_PAD, 1), f32),
            pltpu.VMEM((_ROWS, _COLS), f32),
            pltpu.VMEM((_PAD, 40), f32),
            pltpu.VMEM((_PAD, 8), f32),
            pltpu.VMEM((6 * _NB, _COLS), f32),
            pltpu.VMEM((_ROWS, _COLS), f32),
            pltpu.VMEM((_ROWS, _COLS), f32),
            pltpu.VMEM((_PAD, 1), f32),
        ],
    )(C, s)

    m = keep.reshape(_PAD)[:_N]
    return jnp.concatenate([boxes, scores[:, None]], axis=1) * m[:, None]
